# baseline (speedup 1.0000x reference)
import functools

import jax
import jax.numpy as jnp
from jax import lax
from jax.experimental import pallas as pl
from jax.experimental.pallas import tpu as pltpu
from jax.experimental.pallas import tpu_sc as plsc

_BATCH = 16384
_VIS = 512
_EMB = 8
_HID = 16

_NC = 2
_NS = 16
_NW = _NC * _NS
_BPW = _BATCH // _NW
_CH = 128
_NCH = _BPW // _CH
_IDROWS = _BATCH // _CH


def _sc_gather_body(u2d, i2d, c2d, ub2d, ib2d, cb2d,
                    uemb, ubias, iemb, ibias, cemb, cbias,
                    u_o, i_o, c_o, bu_o, bi_o, bc_o,
                    uidx, iidx, cidx, buidx, biidx, bcidx,
                    ub, ib, cb, bub, bib, bcb, sem):
    c = lax.axis_index("c")
    s = lax.axis_index("s")
    wid = s * _NC + c
    r0 = wid * _NCH
    base = wid * _BPW
    pltpu.sync_copy(u2d.at[pl.ds(r0, _NCH)], uidx)
    pltpu.sync_copy(i2d.at[pl.ds(r0, _NCH)], iidx)
    pltpu.sync_copy(c2d.at[pl.ds(r0, _NCH)], cidx)
    pltpu.sync_copy(ub2d.at[pl.ds(r0, _NCH)], buidx)
    pltpu.sync_copy(ib2d.at[pl.ds(r0, _NCH)], biidx)
    pltpu.sync_copy(cb2d.at[pl.ds(r0, _NCH)], bcidx)
    copies = []
    for ch in range(_NCH):
        sl = pl.ds(ch * _CH, _CH)
        copies.append(pltpu.async_copy(uemb.at[uidx.at[ch]], ub.at[sl], sem))
        copies.append(pltpu.async_copy(iemb.at[iidx.at[ch]], ib.at[sl], sem))
        copies.append(pltpu.async_copy(cemb.at[cidx.at[ch]], cb.at[sl], sem))
        copies.append(pltpu.async_copy(ubias.at[buidx.at[ch]], bub.at[sl], sem))
        copies.append(pltpu.async_copy(ibias.at[biidx.at[ch]], bib.at[sl], sem))
        copies.append(pltpu.async_copy(cbias.at[bcidx.at[ch]], bcb.at[sl], sem))
    for cp in copies:
        cp.wait()
    osl = pl.ds(base, _BPW)
    pltpu.sync_copy(ub, u_o.at[osl])
    pltpu.sync_copy(ib, i_o.at[osl])
    pltpu.sync_copy(cb, c_o.at[osl])
    pltpu.sync_copy(bub, bu_o.at[osl])
    pltpu.sync_copy(bib, bi_o.at[osl])
    pltpu.sync_copy(bcb, bc_o.at[osl])


_sc_gather = functools.partial(
    pl.kernel,
    mesh=plsc.VectorSubcoreMesh(core_axis_name="c", subcore_axis_name="s"),
    out_type=tuple(
        jax.ShapeDtypeStruct((_BATCH, _EMB), jnp.float32) for _ in range(6)),
    scratch_types=(
        [pltpu.VMEM((_NCH, _CH), jnp.int32) for _ in range(6)]
        + [pltpu.VMEM((_BPW, _EMB), jnp.float32) for _ in range(6)]
        + [pltpu.SemaphoreType.DMA]),
    compiler_params=pltpu.CompilerParams(use_tc_tiling_on_sc=False),
)(_sc_gather_body)


_BLK = 2048
_NBLK = _BATCH // _BLK


def _lane_select(rows, idx):
    lane = lax.broadcasted_iota(jnp.int32, rows.shape, 1)
    return jnp.sum(jnp.where(lane == (idx & 7)[:, None], rows, 0.0), axis=1)


def _tc_body(scal_ref, vis_ref, w1_ref, b1_ref, w2_ref, b2_ref,
             u_ref, i_ref, c_ref, bu_ref, bi_ref, bc_ref,
             uid_ref, iid_ref, cid_ref, out_ref):
    w = scal_ref[0, 0]
    vb = scal_ref[0, 1]
    mn = scal_ref[0, 2]
    h = jnp.maximum(
        jnp.dot(vis_ref[...], w1_ref[...], preferred_element_type=jnp.float32)
        + b1_ref[...], 0.0)
    v = jnp.dot(h, w2_ref[...], preferred_element_type=jnp.float32) + b2_ref[...]
    i2 = (1.0 - w) * i_ref[...] + w * (v + c_ref[...])
    bu = _lane_select(bu_ref[...], uid_ref[0, 0, :])
    bi = _lane_select(bi_ref[...], iid_ref[0, 0, :])
    bc = _lane_select(bc_ref[...], cid_ref[0, 0, :])
    pred = (jnp.sum(u_ref[...] * i2, axis=1)
            + bu + bi + w * (vb + bc) + mn)
    out_ref[0, 0, :] = pred


def kernel(u_id, i_id, weight, visual_features, category_features,
           user_emb, user_bias, item_emb, item_bias,
           W1, b1, W2, b2, visual_bias, category_emb, category_bias, mean):
    u_id = u_id.astype(jnp.int32)
    i_id = i_id.astype(jnp.int32)
    cat = category_features.astype(jnp.int32)
    u2d = u_id.reshape(_IDROWS, _CH)
    i2d = i_id.reshape(_IDROWS, _CH)
    c2d = cat.reshape(_IDROWS, _CH)
    ub2d = (u_id >> 3).reshape(_IDROWS, _CH)
    ib2d = (i_id >> 3).reshape(_IDROWS, _CH)
    cb2d = (cat >> 3).reshape(_IDROWS, _CH)

    U, I, C, bup, bip, bcp = _sc_gather(
        u2d, i2d, c2d, ub2d, ib2d, cb2d,
        user_emb, user_bias.reshape(-1, 8),
        item_emb, item_bias.reshape(-1, 8),
        category_emb, category_bias.reshape(-1, 8))

    scal = jnp.concatenate([weight, visual_bias, mean]).reshape(1, 3)
    uid3 = u_id.reshape(_NBLK, 1, _BLK)
    iid3 = i_id.reshape(_NBLK, 1, _BLK)
    cid3 = cat.reshape(_NBLK, 1, _BLK)

    out = pl.pallas_call(
        _tc_body,
        grid=(_NBLK,),
        in_specs=[
            pl.BlockSpec((1, 3), lambda i: (0, 0), memory_space=pltpu.SMEM),
            pl.BlockSpec((_BLK, _VIS), lambda i: (i, 0)),
            pl.BlockSpec((_VIS, _HID), lambda i: (0, 0)),
            pl.BlockSpec((1, _HID), lambda i: (0, 0)),
            pl.BlockSpec((_HID, _EMB), lambda i: (0, 0)),
            pl.BlockSpec((1, _EMB), lambda i: (0, 0)),
            pl.BlockSpec((_BLK, _EMB), lambda i: (i, 0)),
            pl.BlockSpec((_BLK, _EMB), lambda i: (i, 0)),
            pl.BlockSpec((_BLK, _EMB), lambda i: (i, 0)),
            pl.BlockSpec((_BLK, _EMB), lambda i: (i, 0)),
            pl.BlockSpec((_BLK, _EMB), lambda i: (i, 0)),
            pl.BlockSpec((_BLK, _EMB), lambda i: (i, 0)),
            pl.BlockSpec((1, 1, _BLK), lambda i: (i, 0, 0)),
            pl.BlockSpec((1, 1, _BLK), lambda i: (i, 0, 0)),
            pl.BlockSpec((1, 1, _BLK), lambda i: (i, 0, 0)),
        ],
        out_specs=pl.BlockSpec((1, 1, _BLK), lambda i: (i, 0, 0)),
        out_shape=jax.ShapeDtypeStruct((_NBLK, 1, _BLK), jnp.float32),
    )(scal, visual_features, W1, b1.reshape(1, _HID), W2, b2.reshape(1, _EMB),
      U, I, C, bup, bip, bcp, uid3, iid3, cid3)

    return out.reshape(_BATCH)
